# baseline (speedup 1.0000x reference)
import jax
import jax.numpy as jnp
from jax.experimental import pallas as pl

THRESH = 0.1

_TN = (((0,), (0,)), ((), ()))
_NN = (((1,), (0,)), ((), ()))


def _dot(a, b, dims):
    return jax.lax.dot_general(a, b, dims, preferred_element_type=jnp.float32)


def _normalize_cols(x):
    n = jnp.maximum(jnp.sqrt(jnp.sum(x * x, axis=0, keepdims=True)), 1e-8)
    return x / n


def _hg_kernel(xt_ref, xc1_ref, xc2_ref, w1e_ref, b1e_ref, w2e_ref, b2e_ref,
               yt_ref, yc1_ref, yc2_ref):
    w1e = w1e_ref[...]
    b1e = b1e_ref[...]
    w2e = w2e_ref[...]
    b2e = b2e_ref[...]
    nc = w1e.shape[0]

    selfs = (xt_ref[0], xc2_ref[0])
    nbrs = ((xt_ref[1], xc1_ref[0]), (xc1_ref[1], xc2_ref[1]))

    for i in range(2):
        tn = _normalize_cols(xt_ref[i])
        cab = jnp.concatenate([_normalize_cols(xc1_ref[i]),
                               _normalize_cols(xc2_ref[i])], axis=1)

        m = (_dot(tn, cab, _TN) > THRESH).astype(jnp.float32)
        mt = (_dot(cab, tn, _TN) > THRESH).astype(jnp.float32)

        x_self = selfs[i]
        x_nab = jnp.concatenate([nbrs[i][0], nbrs[i][1]], axis=1)
        t_self = _dot(w1e, x_self, _TN) + b1e
        t_nab = _dot(w1e, x_nab, _TN) + b1e

        s = t_self + _dot(t_nab, mt, _NN)
        x_edge = s[:nc] / s[nc:nc + 1]
        u = _dot(w2e, x_edge, _TN) + b2e

        stu = _dot(u, m, _NN)
        s_ab = stu[:nc] / jnp.maximum(stu[nc:nc + 1], 1.0)

        n = tn.shape[1]
        if i == 0:
            yt_ref[0] = u[:nc]
            yt_ref[1] = s_ab[:, :n]
            yc1_ref[0] = s_ab[:, n:]
        else:
            yc2_ref[0] = u[:nc]
            yc1_ref[1] = s_ab[:, :n]
            yc2_ref[1] = s_ab[:, n:]


def kernel(X_target, X_context1, X_context2, W1, b1, W2, b2):
    B, C, Hh, Ww = X_target.shape
    N = Hh * Ww
    xt = X_target.reshape(B, C, N)
    xc1 = X_context1.reshape(B, C, N)
    xc2 = X_context2.reshape(B, C, N)

    zcol = jnp.zeros((C, 1), jnp.float32)
    one = jnp.ones((1, 1), jnp.float32)
    w1e = jnp.concatenate([W1, zcol], axis=1)
    w2e = jnp.concatenate([W2, zcol], axis=1)
    b1e = jnp.concatenate([b1.reshape(C, 1), one], axis=0)
    b2e = jnp.concatenate([b2.reshape(C, 1), one], axis=0)

    shp = jax.ShapeDtypeStruct((B, C, N), jnp.float32)
    yt, yc1, yc2 = pl.pallas_call(
        _hg_kernel,
        out_shape=[shp, shp, shp],
    )(xt, xc1, xc2, w1e, b1e, w2e, b2e)

    rs = lambda a: a.reshape(B, C, Hh, Ww)
    return (rs(yt), rs(yc1), rs(yc2))


# speedup vs baseline: 1.0081x; 1.0081x over previous
import jax
import jax.numpy as jnp
from jax.experimental import pallas as pl

THRESH = 0.1

_TN = (((0,), (0,)), ((), ()))
_NN = (((1,), (0,)), ((), ()))


def _dot(a, b, dims):
    return jax.lax.dot_general(a, b, dims, preferred_element_type=jnp.float32)


def _normalize_cols(x):
    n = jnp.maximum(jnp.sqrt(jnp.sum(x * x, axis=0, keepdims=True)), 1e-8)
    return x / n


def _hg_kernel(xt_ref, xc1_ref, xc2_ref, w1_ref, b1_ref, w2_ref, b2_ref,
               yt_ref, yc1_ref, yc2_ref):
    i = pl.program_id(0)
    w1 = w1_ref[...]
    b1 = b1_ref[...]
    w2 = w2_ref[...]
    b2 = b2_ref[...]
    first = i == 0

    xt_i = jnp.where(first, xt_ref[0], xt_ref[1])
    tn = _normalize_cols(xt_i)
    ca = _normalize_cols(xc1_ref[0])
    cb = _normalize_cols(jnp.where(first, xc2_ref[0], xc2_ref[1]))

    m_a = (_dot(tn, ca, _TN) > THRESH).astype(jnp.float32)
    m_b = (_dot(tn, cb, _TN) > THRESH).astype(jnp.float32)
    mt_a = (_dot(ca, tn, _TN) > THRESH).astype(jnp.float32)
    mt_b = (_dot(cb, tn, _TN) > THRESH).astype(jnp.float32)

    d_e = (1.0 + jnp.sum(mt_a, axis=0, keepdims=True)
           + jnp.sum(mt_b, axis=0, keepdims=True))

    x_self = jnp.where(first, xt_ref[0], xc2_ref[0])
    x_na = jnp.where(first, xt_ref[1], xc1_ref[0])
    x_nb = jnp.where(first, xc1_ref[0], xc2_ref[1])

    t_self = _dot(w1, x_self, _TN) + b1
    t_na = _dot(w1, x_na, _TN) + b1
    t_nb = _dot(w1, x_nb, _TN) + b1

    x_edge = (t_self + _dot(t_na, mt_a, _NN) + _dot(t_nb, mt_b, _NN)) / d_e
    u = _dot(w2, x_edge, _TN) + b2

    d_va = jnp.maximum(jnp.sum(m_a, axis=0, keepdims=True), 1.0)
    d_vb = jnp.maximum(jnp.sum(m_b, axis=0, keepdims=True), 1.0)
    s_a = _dot(u, m_a, _NN) / d_va
    s_b = _dot(u, m_b, _NN) / d_vb

    @pl.when(first)
    def _():
        yt_ref[0] = u
        yt_ref[1] = s_a
        yc1_ref[0] = s_b

    @pl.when(jnp.logical_not(first))
    def _():
        yc2_ref[0] = u
        yc1_ref[1] = s_a
        yc2_ref[1] = s_b


def kernel(X_target, X_context1, X_context2, W1, b1, W2, b2):
    B, C, Hh, Ww = X_target.shape
    N = Hh * Ww
    xt = X_target.reshape(B, C, N)
    xc1 = X_context1.reshape(B, C, N)
    xc2 = X_context2.reshape(B, C, N)
    b1c = b1.reshape(C, 1)
    b2c = b2.reshape(C, 1)

    shp = jax.ShapeDtypeStruct((B, C, N), jnp.float32)
    full = lambda: pl.BlockSpec((B, C, N), lambda i: (0, 0, 0))
    yt, yc1, yc2 = pl.pallas_call(
        _hg_kernel,
        grid=(B,),
        in_specs=[
            full(),
            pl.BlockSpec((1, C, N), lambda i: (i, 0, 0)),
            full(),
            pl.BlockSpec((C, C), lambda i: (0, 0)),
            pl.BlockSpec((C, 1), lambda i: (0, 0)),
            pl.BlockSpec((C, C), lambda i: (0, 0)),
            pl.BlockSpec((C, 1), lambda i: (0, 0)),
        ],
        out_specs=[full(), full(), full()],
        out_shape=[shp, shp, shp],
    )(xt, xc1, xc2, W1, b1c, W2, b2c)

    rs = lambda a: a.reshape(B, C, Hh, Ww)
    return (rs(yt), rs(yc1), rs(yc2))


# speedup vs baseline: 1.0646x; 1.0561x over previous
import jax
import jax.numpy as jnp
from jax.experimental import pallas as pl

THRESH = 0.1

_TN = (((0,), (0,)), ((), ()))
_NN = (((1,), (0,)), ((), ()))


def _dot(a, b, dims):
    return jax.lax.dot_general(a, b, dims, preferred_element_type=jnp.float32)


def _normalize_cols(x):
    n = jnp.maximum(jnp.sqrt(jnp.sum(x * x, axis=0, keepdims=True)), 1e-8)
    return x / n


def _hg_kernel(xt_ref, xc1_ref, xc2_ref, w1_ref, b1_ref, w2_ref, b2_ref,
               yt_ref, yc1_ref, yc2_ref):
    w1 = w1_ref[...]
    b1 = b1_ref[...]
    w2 = w2_ref[...]
    b2 = b2_ref[...]

    selfs = (xt_ref[0], xc2_ref[0])
    nbrs = ((xt_ref[1], xc1_ref[0]), (xc1_ref[1], xc2_ref[1]))

    for i in range(2):
        tn = _normalize_cols(xt_ref[i])
        ca = _normalize_cols(xc1_ref[i])
        cb = _normalize_cols(xc2_ref[i])

        m_a = (_dot(tn, ca, _TN) > THRESH).astype(jnp.float32)
        m_b = (_dot(tn, cb, _TN) > THRESH).astype(jnp.float32)
        mt_a = m_a.T
        mt_b = m_b.T

        d_e = (1.0 + jnp.sum(mt_a, axis=0, keepdims=True)
               + jnp.sum(mt_b, axis=0, keepdims=True))

        t_self = _dot(w1, selfs[i], _TN) + b1
        t_na = _dot(w1, nbrs[i][0], _TN) + b1
        t_nb = _dot(w1, nbrs[i][1], _TN) + b1

        x_edge = (t_self + _dot(t_na, mt_a, _NN) + _dot(t_nb, mt_b, _NN)) / d_e
        u = _dot(w2, x_edge, _TN) + b2

        d_va = jnp.maximum(jnp.sum(m_a, axis=0, keepdims=True), 1.0)
        d_vb = jnp.maximum(jnp.sum(m_b, axis=0, keepdims=True), 1.0)
        s_a = _dot(u, m_a, _NN) / d_va
        s_b = _dot(u, m_b, _NN) / d_vb

        if i == 0:
            yt_ref[0] = u
            yt_ref[1] = s_a
            yc1_ref[0] = s_b
        else:
            yc2_ref[0] = u
            yc1_ref[1] = s_a
            yc2_ref[1] = s_b


def kernel(X_target, X_context1, X_context2, W1, b1, W2, b2):
    B, C, Hh, Ww = X_target.shape
    N = Hh * Ww
    xt = X_target.reshape(B, C, N)
    xc1 = X_context1.reshape(B, C, N)
    xc2 = X_context2.reshape(B, C, N)
    b1c = b1.reshape(C, 1)
    b2c = b2.reshape(C, 1)

    shp = jax.ShapeDtypeStruct((B, C, N), jnp.float32)
    yt, yc1, yc2 = pl.pallas_call(
        _hg_kernel,
        out_shape=[shp, shp, shp],
    )(xt, xc1, xc2, W1, b1c, W2, b2c)

    rs = lambda a: a.reshape(B, C, Hh, Ww)
    return (rs(yt), rs(yc1), rs(yc2))
